# Initial kernel scaffold; baseline (speedup 1.0000x reference)
import jax
import jax.numpy as jnp
from jax.experimental import pallas as pl


def kernel(v, one_hop_list, two_hop_list, doc_topic_dist, word_topic_dist):
    raise NotImplementedError("write your pallas kernel here")


# speedup vs baseline: 11.0448x; 11.0448x over previous
import functools

import jax
import jax.numpy as jnp
from jax import lax
from jax.experimental import pallas as pl
from jax.experimental.pallas import tpu as pltpu
from jax.experimental.pallas import tpu_sc as plsc

_K = 128
_L = 16
_NJ = _K // _L
_ONE_HOP = 32
_TWO_HOP = 64
_DROWS = 1 + _TWO_HOP
_NC = 2
_NS = 16
_NW = _NC * _NS


def _permute(x, idx):
    return lax.gather(
        x, idx[:, None],
        lax.GatherDimensionNumbers(
            offset_dims=(), collapsed_slice_dims=(0,), start_index_map=(0,)),
        (1,), mode=lax.GatherScatterMode.PROMISE_IN_BOUNDS)


def _combine_row(dr, wr, g, out_v):
    inv1 = 1.0 / _ONE_HOP
    inv2 = 1.0 / _TWO_HOP

    def acc_doc(r, acc):
        return [acc[j] + dr[r, pl.ds(j * _L, _L)] for j in range(_NJ)]

    def acc_word(r, acc):
        return [acc[j] + wr[r, pl.ds(j * _L, _L)] for j in range(_NJ)]

    two = lax.fori_loop(
        2, _DROWS, acc_doc,
        [dr[1, pl.ds(j * _L, _L)] for j in range(_NJ)], unroll=4)
    one = lax.fori_loop(
        1, _ONE_HOP, acc_word,
        [wr[0, pl.ds(j * _L, _L)] for j in range(_NJ)], unroll=4)
    t = [dr[0, pl.ds(j * _L, _L)] + two[j] * inv2 + one[j] * inv1
         for j in range(_NJ)]

    m16 = t[0]
    for j in range(1, _NJ):
        m16 = jnp.maximum(m16, t[j])
    lanes = lax.iota(jnp.int32, _L)
    for k in (8, 4, 2, 1):
        m16 = jnp.maximum(m16, _permute(m16, lanes ^ k))
    e = [jnp.exp(t[j] - m16) for j in range(_NJ)]
    s16 = e[0]
    for j in range(1, _NJ):
        s16 = s16 + e[j]
    for k in (8, 4, 2, 1):
        s16 = s16 + _permute(s16, lanes ^ k)
    r = 1.0 / s16
    for j in range(_NJ):
        out_v[g, pl.ds(j * _L, _L)] = e[j] * r


def kernel(v, one_hop_list, two_hop_list, doc_topic_dist, word_topic_dist):
    B = v.shape[0]
    assert B % _NW == 0
    ipw = B // _NW

    doc_idx = jnp.concatenate(
        [v.astype(jnp.int32)[:, None], two_hop_list.astype(jnp.int32)], axis=1)
    word_idx = one_hop_list.astype(jnp.int32)

    mesh = plsc.VectorSubcoreMesh(
        core_axis_name="c", subcore_axis_name="s",
        num_cores=_NC, num_subcores=_NS)

    @functools.partial(
        pl.kernel,
        out_type=jax.ShapeDtypeStruct((B, _K), jnp.float32),
        mesh=mesh,
        scratch_types=[
            pltpu.VMEM((ipw, _DROWS), jnp.int32),
            pltpu.VMEM((ipw, _ONE_HOP), jnp.int32),
            pltpu.VMEM((2, _DROWS, _K), jnp.float32),
            pltpu.VMEM((2, _ONE_HOP, _K), jnp.float32),
            pltpu.VMEM((ipw, _K), jnp.float32),
            pltpu.SemaphoreType.DMA,
            pltpu.SemaphoreType.DMA,
            pltpu.SemaphoreType.DMA,
            pltpu.SemaphoreType.DMA,
        ],
    )
    def run(doc_tab, word_tab, didx_hbm, widx_hbm, out_hbm,
            didx_v, widx_v, drows, wrows, out_v,
            dsem0, dsem1, wsem0, wsem1):
        wid = lax.axis_index("s") * _NC + lax.axis_index("c")
        base = wid * ipw
        pltpu.sync_copy(didx_hbm.at[pl.ds(base, ipw)], didx_v)
        pltpu.sync_copy(widx_hbm.at[pl.ds(base, ipw)], widx_v)

        dsems = (dsem0, dsem1)
        wsems = (wsem0, wsem1)

        def issue(g, slot):
            pltpu.async_copy(doc_tab.at[didx_v.at[g]], drows.at[slot],
                             dsems[slot])
            pltpu.async_copy(word_tab.at[widx_v.at[g]], wrows.at[slot],
                             wsems[slot])

        def wait(g, slot):
            pltpu.make_async_copy(doc_tab.at[didx_v.at[g]], drows.at[slot],
                                  dsems[slot]).wait()
            pltpu.make_async_copy(word_tab.at[widx_v.at[g]], wrows.at[slot],
                                  wsems[slot]).wait()

        issue(0, 0)
        issue(1, 1)

        def pair(p, carry):
            for b in range(2):
                g = p * 2 + b
                wait(g, b)
                _combine_row(drows.at[b], wrows.at[b], g, out_v)

                @pl.when(g + 2 < ipw)
                def _prefetch(b=b, g=g):
                    issue(g + 2, b)
            return carry

        lax.fori_loop(0, ipw // 2, pair, 0)
        pltpu.sync_copy(out_v, out_hbm.at[pl.ds(base, ipw)])

    return run(doc_topic_dist, word_topic_dist, doc_idx, word_idx)
